# baseline (speedup 1.0000x reference)
import functools

import jax
import jax.numpy as jnp
from jax import lax
from jax.experimental import pallas as pl
from jax.experimental.pallas import tpu as pltpu
from jax.experimental.pallas import tpu_sc as plsc

N = 10000
E = 320000
D = 128

_F32 = jnp.float32


def _acosh(z):
    return jnp.log(z + jnp.sqrt(z * z - 1.0))


def _cosh_sinh(n):
    e = jnp.exp(n)
    ei = 1.0 / e
    return 0.5 * (e + ei), 0.5 * (e - ei)


def _sigmoid(z):
    return 1.0 / (1.0 + jnp.exp(-z))


def _rowsum(z):
    return jnp.sum(z, axis=-1, keepdims=True)


def _first_mask(shape):
    return lax.broadcasted_iota(jnp.int32, shape, len(shape) - 1) == 0


def _logmap0_cols(x, first):
    x0 = jnp.maximum(x[:, 0:1], 1.0 + 1e-7)
    d = _acosh(x0)
    nsq = _rowsum(x * x) - x[:, 0:1] * x[:, 0:1]
    n = jnp.sqrt(jnp.maximum(nsq, 1e-12))
    return jnp.where(first, 0.0, (d / n) * x)


def _expmap0_cols(u, first):
    us = jnp.where(first, 0.0, u)
    nsq = _rowsum(us * us)
    n = jnp.sqrt(jnp.maximum(nsq, 1e-12))
    c, s = _cosh_sinh(n)
    return jnp.where(first, c, (s / n) * us)


def _bf16_bits(x):
    u = lax.bitcast_convert_type(x, jnp.int32)
    return lax.shift_right_logical(
        u + 0x7FFF + (lax.shift_right_logical(u, 16) & 1), 16)


def _pack2(lo, hi):
    return _bf16_bits(lo) | (_bf16_bits(hi) << 16)


def _unpack_lo(w):
    return lax.bitcast_convert_type(w << 16, _F32)


def _unpack_hi(w):
    return lax.bitcast_convert_type(w & jnp.int32(-65536), _F32)


def _prologue_body(h_ref, wt_ref, bias_ref, w1at_ref, w1bt_ref,
                   x_ref, tr_ref, tc_ref):
    h = h_ref[...]
    first = _first_mask(h.shape)
    u = _logmap0_cols(h, first)
    xt = jnp.dot(u, wt_ref[...], preferred_element_type=_F32)
    xt = jnp.where(first, 0.0, xt)
    x = _expmap0_cols(xt, first)
    bmask = jnp.where(_first_mask(bias_ref[...].shape), 0.0, bias_ref[...])
    c = _rowsum(x * bmask) / (1.0 + x[:, 0:1])
    b = bmask + c * (jnp.where(first, 1.0, 0.0) + x)
    lbb = _rowsum(b * b) - 2.0 * b[:, 0:1] * b[:, 0:1]
    nb = jnp.sqrt(jnp.maximum(lbb, 1e-12))
    ch, sh = _cosh_sinh(nb)
    x = ch * x + (sh / nb) * b
    x_ref[...] = x
    x_tan = _logmap0_cols(x, first)
    a = jnp.dot(x_tan, w1at_ref[...], preferred_element_type=_F32)
    bb = jnp.dot(x_tan, w1bt_ref[...], preferred_element_type=_F32)
    tr_ref[...] = _pack2(x, a)
    tc_ref[...] = _pack2(x, bb)


def _edge_body(gr_ref, gc_ref, ea_ref, em_ref, wv_ref, agg_ref):
    wr = gr_ref[...]
    wc = gc_ref[...]
    xr = _unpack_lo(wr)
    ar = _unpack_hi(wr)
    xc = _unpack_lo(wc)
    bc = _unpack_hi(wc)
    alpha = -(_rowsum(xr * xc) - 2.0 * xr[:, 0:1] * xc[:, 0:1])
    alpha = jnp.maximum(alpha, 1.0 + 1e-7)
    al = alpha.reshape(_BE // D, D)
    sq = jnp.sqrt(jnp.maximum(al * al - 1.0, 1e-12))
    dal = jnp.log(al + sq)
    fac = dal / sq
    d = dal.reshape(_BE, 1)
    w1c0 = wv_ref[0:1, :]
    w1c1 = wv_ref[1:2, :]
    b1 = wv_ref[2:3, :]
    w2 = wv_ref[3:4, :]
    b2 = wv_ref[4:5, 0:1]
    pre = ar + bc + ea_ref[...] * w1c0 + d * w1c1 + b1
    hmid = pre * _sigmoid(pre)
    lg = _rowsum(hmid * w2) + b2
    att_fac = _sigmoid(lg.reshape(_BE // D, D)) * fac
    fa = att_fac.reshape(_BE, 1) * em_ref[...]
    agg_ref[...] = fa * (xc - alpha * xr)


def _epilogue_body(x_ref, o0_ref, o1_ref, ln_ref, out_ref):
    x = x_ref[...]
    first = _first_mask(x.shape)
    out = (o0_ref[...] + o1_ref[...]) * (1.0 / 1000.0)
    lxo = _rowsum(x * out) - 2.0 * x[:, 0:1] * out[:, 0:1]
    p = out + lxo * x
    lpp = _rowsum(p * p) - 2.0 * p[:, 0:1] * p[:, 0:1]
    npn = jnp.sqrt(jnp.maximum(lpp, 1e-12))
    ch, sh = _cosh_sinh(npn)
    x2 = ch * x + (sh / npn) * p
    ht = _logmap0_cols(x2, first)
    mu = _rowsum(ht) * (1.0 / (D - 1))
    dif = jnp.where(first, 0.0, ht - mu)
    var = _rowsum(dif * dif) * (1.0 / (D - 1))
    gamma = ln_ref[0:1, :]
    beta = ln_ref[1:2, :]
    h1 = dif / jnp.sqrt(var + 1e-5) * gamma + beta
    x3 = _expmap0_cols(h1, first)
    s = _logmap0_cols(x3, first)
    sl = s * _sigmoid(s)
    out_ref[...] = _expmap0_cols(sl, first)


_NC = 2
_NS = 16
_NW = _NC * _NS
_EPERW = 10240
_EPAD = _NW * _EPERW
_S = 8
_ESL = _EPAD // _S
_EPWS = _ESL // _NW
_CHG = 80
_NCHG = _EPWS // _CHG
_CHS = 128
_NCHS = _EPWS // _CHS
_NPAD = 10240
_ROWS_PER_TILE = _NPAD // _NS


def _sc_gather(tr, tcb, row3d, col3d):
    mesh = plsc.VectorSubcoreMesh(core_axis_name="c", subcore_axis_name="s")

    @functools.partial(
        pl.kernel, mesh=mesh,
        out_type=[jax.ShapeDtypeStruct((_ESL, D), jnp.int32),
                  jax.ShapeDtypeStruct((_ESL, D), jnp.int32)],
        scratch_types=[
            pltpu.VMEM((_NCHG, _CHG), jnp.int32),
            pltpu.VMEM((_NCHG, _CHG), jnp.int32),
            pltpu.VMEM((_CHG, D), jnp.int32),
            pltpu.VMEM((_CHG, D), jnp.int32),
            pltpu.VMEM((_CHG, D), jnp.int32),
            pltpu.VMEM((_CHG, D), jnp.int32),
            pltpu.SemaphoreType.DMA,
            pltpu.SemaphoreType.DMA,
            pltpu.SemaphoreType.DMA,
            pltpu.SemaphoreType.DMA,
        ],
    )
    def k(tr_hbm, tc_hbm, row_hbm, col_hbm, gr_hbm, gc_hbm,
          idxr, idxc, br0, br1, bc0, bc1, sr0, sr1, sc0, sc1):
        cid = lax.axis_index("c")
        sid = lax.axis_index("s")
        wid = sid * _NC + cid
        bufr = (br0, br1)
        bufc = (bc0, bc1)
        semr = (sr0, sr1)
        semc = (sc0, sc1)
        pltpu.sync_copy(row_hbm.at[wid], idxr)
        pltpu.sync_copy(col_hbm.at[wid], idxc)

        for b in range(2):
            pltpu.async_copy(tr_hbm.at[idxr.at[b]], bufr[b], semr[b])
            pltpu.async_copy(tc_hbm.at[idxc.at[b]], bufc[b], semc[b])

        def body(g, carry):
            for b in range(2):
                j = g * 2 + b
                ebase = pl.multiple_of(wid * _EPWS + j * _CHG, 8)
                pltpu.make_async_copy(tr_hbm.at[idxr.at[j]], bufr[b],
                                      semr[b]).wait()
                pltpu.sync_copy(bufr[b], gr_hbm.at[pl.ds(ebase, _CHG)])
                pltpu.make_async_copy(tc_hbm.at[idxc.at[j]], bufc[b],
                                      semc[b]).wait()
                pltpu.sync_copy(bufc[b], gc_hbm.at[pl.ds(ebase, _CHG)])

                @pl.when(j + 2 < _NCHG)
                def _():
                    pltpu.async_copy(tr_hbm.at[idxr.at[j + 2]], bufr[b],
                                     semr[b])
                    pltpu.async_copy(tc_hbm.at[idxc.at[j + 2]], bufc[b],
                                     semc[b])
            return carry

        lax.fori_loop(0, _NCHG // 2, body, 0)

    return k(tr, tcb, row3d, col3d)


def _sc_scatter(aggs, row3d, zeros_hbm):
    mesh = plsc.VectorSubcoreMesh(core_axis_name="c", subcore_axis_name="s")

    @functools.partial(
        pl.kernel, mesh=mesh,
        out_type=jax.ShapeDtypeStruct((_NC, _NPAD, D), _F32),
        scratch_types=[
            pltpu.VMEM((_S * _NCHS, _CHS), jnp.int32),
            pltpu.VMEM((_CHS, D), _F32),
            pltpu.VMEM((_CHS, D), _F32),
            pltpu.VMEM_SHARED((_NPAD, D), _F32),
            pltpu.SemaphoreType.DMA,
            pltpu.SemaphoreType.DMA,
        ],
    )
    def k(*args):
        agg_refs = args[:_S]
        (row_hbm, zeros_h, parts_hbm, idxr, b0, b1, acc, sm0, sm1) = args[_S:]
        cid = lax.axis_index("c")
        sid = lax.axis_index("s")
        wid = sid * _NC + cid
        buf = (b0, b1)
        sems = (sm0, sm1)
        pltpu.sync_copy(zeros_h, b0)

        def zbody(t, carry):
            rbase = pl.multiple_of(sid * _ROWS_PER_TILE + t * _CHS, 8)
            pltpu.sync_copy(b0, acc.at[pl.ds(rbase, _CHS)])
            return carry

        lax.fori_loop(0, _ROWS_PER_TILE // _CHS, zbody, 0)
        plsc.subcore_barrier()

        pltpu.sync_copy(row_hbm.at[wid], idxr)

        def ebase_of(j):
            return pl.multiple_of(wid * _EPWS + j * _CHS, 8)

        for s in range(_S):
            agg_hbm = agg_refs[s]
            for b in range(2):
                pltpu.async_copy(agg_hbm.at[pl.ds(ebase_of(b), _CHS)],
                                 buf[b], sems[b])

            def body(g, carry, agg_hbm=agg_hbm, s=s):
                for b in range(2):
                    j = g * 2 + b
                    pltpu.make_async_copy(
                        agg_hbm.at[pl.ds(ebase_of(j), _CHS)],
                        buf[b], sems[b]).wait()
                    pltpu.sync_copy(buf[b], acc.at[idxr.at[s * _NCHS + j]],
                                    add=True)

                    @pl.when(j + 2 < _NCHS)
                    def _():
                        pltpu.async_copy(
                            agg_hbm.at[pl.ds(ebase_of(j + 2), _CHS)],
                            buf[b], sems[b])
                return carry

            lax.fori_loop(0, _NCHS // 2, body, 0)
        plsc.subcore_barrier()

        def wbody(t, carry):
            rbase = pl.multiple_of(sid * _ROWS_PER_TILE + t * _CHS, 8)
            pltpu.sync_copy(acc.at[pl.ds(rbase, _CHS)], b0)
            pltpu.sync_copy(b0, parts_hbm.at[cid].at[pl.ds(rbase, _CHS)])
            return carry

        lax.fori_loop(0, _ROWS_PER_TILE // _CHS, wbody, 0)

    return k(*aggs, row3d, zeros_hbm)


_BN = 2000
_BE = 2048


def kernel(h, edge_attr, edges, node_mask, edge_mask, W, bias, att_w1,
           att_b1, att_w2, att_b2, ln_gamma, ln_beta):
    del node_mask
    f32 = _F32
    h = h.astype(f32)
    npad = _EPAD - E
    pad_idx = (jnp.arange(npad, dtype=jnp.int32) * 13) % N
    row = jnp.concatenate([edges[0].astype(jnp.int32), pad_idx])
    col = jnp.concatenate([edges[1].astype(jnp.int32), pad_idx])
    ea_p = jnp.concatenate([edge_attr.astype(f32),
                            jnp.zeros((npad, 1), f32)])
    em_p = jnp.concatenate([edge_mask.astype(f32),
                            jnp.zeros((npad, 1), f32)])
    row_g = row.reshape(_S, _NW, _NCHG, _CHG)
    col_g = col.reshape(_S, _NW, _NCHG, _CHG)
    row_sc = (row.reshape(_S, _NW, _NCHS, _CHS)
              .transpose(1, 0, 2, 3).reshape(_NW, _S * _NCHS, _CHS))
    ea4 = ea_p.reshape(_S, _ESL, 1)
    em4 = em_p.reshape(_S, _ESL, 1)

    wt = W.T.astype(f32)
    w1at = att_w1[:, 0:D].T.astype(f32)
    w1bt = att_w1[:, D:2 * D].T.astype(f32)
    bias2 = bias.astype(f32).reshape(1, D)

    wv = jnp.zeros((8, D), f32)
    wv = wv.at[0].set(att_w1[:, 2 * D])
    wv = wv.at[1].set(att_w1[:, 2 * D + 1])
    wv = wv.at[2].set(att_b1)
    wv = wv.at[3].set(att_w2[0])
    wv = wv.at[4, 0].set(att_b2[0])

    ln = jnp.zeros((2, D), f32)
    ln = ln.at[0, 1:].set(ln_gamma)
    ln = ln.at[1, 1:].set(ln_beta)

    nblk = N // _BN
    x, tr, tcb = pl.pallas_call(
        _prologue_body,
        grid=(nblk,),
        in_specs=[
            pl.BlockSpec((_BN, D), lambda i: (i, 0)),
            pl.BlockSpec((D, D), lambda i: (0, 0)),
            pl.BlockSpec((1, D), lambda i: (0, 0)),
            pl.BlockSpec((D, D), lambda i: (0, 0)),
            pl.BlockSpec((D, D), lambda i: (0, 0)),
        ],
        out_specs=[
            pl.BlockSpec((_BN, D), lambda i: (i, 0)),
            pl.BlockSpec((_BN, D), lambda i: (i, 0)),
            pl.BlockSpec((_BN, D), lambda i: (i, 0)),
        ],
        out_shape=[
            jax.ShapeDtypeStruct((N, D), f32),
            jax.ShapeDtypeStruct((N, D), jnp.int32),
            jax.ShapeDtypeStruct((N, D), jnp.int32),
        ],
    )(h, wt, bias2, w1at, w1bt)

    eblk = _ESL // _BE
    aggs = []
    for s in range(_S):
        gr, gc = _sc_gather(tr, tcb, row_g[s], col_g[s])
        agg_s = pl.pallas_call(
            _edge_body,
            grid=(eblk,),
            in_specs=[
                pl.BlockSpec((_BE, D), lambda i: (i, 0)),
                pl.BlockSpec((_BE, D), lambda i: (i, 0)),
                pl.BlockSpec((_BE, 1), lambda i: (i, 0)),
                pl.BlockSpec((_BE, 1), lambda i: (i, 0)),
                pl.BlockSpec((8, D), lambda i: (0, 0)),
            ],
            out_specs=pl.BlockSpec((_BE, D), lambda i: (i, 0)),
            out_shape=jax.ShapeDtypeStruct((_ESL, D), f32),
        )(gr, gc, ea4[s], em4[s], wv)
        aggs.append(agg_s)

    zeros_h = jnp.zeros((_CHS, D), f32)
    parts = _sc_scatter(aggs, row_sc, zeros_h)
    p0 = parts[0, :N]
    p1 = parts[1, :N]

    out = pl.pallas_call(
        _epilogue_body,
        grid=(nblk,),
        in_specs=[
            pl.BlockSpec((_BN, D), lambda i: (i, 0)),
            pl.BlockSpec((_BN, D), lambda i: (i, 0)),
            pl.BlockSpec((_BN, D), lambda i: (i, 0)),
            pl.BlockSpec((2, D), lambda i: (0, 0)),
        ],
        out_specs=pl.BlockSpec((_BN, D), lambda i: (i, 0)),
        out_shape=jax.ShapeDtypeStruct((N, D), f32),
    )(x, p0, p1, ln)

    return out


# speedup vs baseline: 1.0287x; 1.0287x over previous
import functools

import jax
import jax.numpy as jnp
from jax import lax
from jax.experimental import pallas as pl
from jax.experimental.pallas import tpu as pltpu
from jax.experimental.pallas import tpu_sc as plsc

N = 10000
E = 320000
D = 128

_F32 = jnp.float32


def _acosh(z):
    return jnp.log(z + jnp.sqrt(z * z - 1.0))


def _cosh_sinh(n):
    e = jnp.exp(n)
    ei = 1.0 / e
    return 0.5 * (e + ei), 0.5 * (e - ei)


def _sigmoid(z):
    return 1.0 / (1.0 + jnp.exp(-z))


def _rowsum(z):
    return jnp.sum(z, axis=-1, keepdims=True)


def _first_mask(shape):
    return lax.broadcasted_iota(jnp.int32, shape, len(shape) - 1) == 0


def _logmap0_cols(x, first):
    x0 = jnp.maximum(x[:, 0:1], 1.0 + 1e-7)
    d = _acosh(x0)
    nsq = _rowsum(x * x) - x[:, 0:1] * x[:, 0:1]
    n = jnp.sqrt(jnp.maximum(nsq, 1e-12))
    return jnp.where(first, 0.0, (d / n) * x)


def _expmap0_cols(u, first):
    us = jnp.where(first, 0.0, u)
    nsq = _rowsum(us * us)
    n = jnp.sqrt(jnp.maximum(nsq, 1e-12))
    c, s = _cosh_sinh(n)
    return jnp.where(first, c, (s / n) * us)


def _bf16_bits(x):
    u = lax.bitcast_convert_type(x, jnp.int32)
    return lax.shift_right_logical(
        u + 0x7FFF + (lax.shift_right_logical(u, 16) & 1), 16)


def _pack2(lo, hi):
    return _bf16_bits(lo) | (_bf16_bits(hi) << 16)


def _unpack_lo(w):
    return lax.bitcast_convert_type(w << 16, _F32)


def _unpack_hi(w):
    return lax.bitcast_convert_type(w & jnp.int32(-65536), _F32)


def _prologue_body(h_ref, wt_ref, bias_ref, w1at_ref, w1bt_ref,
                   x_ref, tr_ref, tc_ref):
    h = h_ref[...]
    first = _first_mask(h.shape)
    u = _logmap0_cols(h, first)
    xt = jnp.dot(u, wt_ref[...], preferred_element_type=_F32)
    xt = jnp.where(first, 0.0, xt)
    x = _expmap0_cols(xt, first)
    bmask = jnp.where(_first_mask(bias_ref[...].shape), 0.0, bias_ref[...])
    c = _rowsum(x * bmask) / (1.0 + x[:, 0:1])
    b = bmask + c * (jnp.where(first, 1.0, 0.0) + x)
    lbb = _rowsum(b * b) - 2.0 * b[:, 0:1] * b[:, 0:1]
    nb = jnp.sqrt(jnp.maximum(lbb, 1e-12))
    ch, sh = _cosh_sinh(nb)
    x = ch * x + (sh / nb) * b
    x_ref[...] = x
    x_tan = _logmap0_cols(x, first)
    a = jnp.dot(x_tan, w1at_ref[...], preferred_element_type=_F32)
    bb = jnp.dot(x_tan, w1bt_ref[...], preferred_element_type=_F32)
    tr_ref[...] = _pack2(x, a)
    tc_ref[...] = _pack2(x, bb)


def _edge_body(gr_ref, gc_ref, ea_ref, em_ref, wv_ref, agg_ref):
    wr = gr_ref[...]
    wc = gc_ref[...]
    xr = _unpack_lo(wr)
    ar = _unpack_hi(wr)
    xc = _unpack_lo(wc)
    bc = _unpack_hi(wc)
    alpha = -(_rowsum(xr * xc) - 2.0 * xr[:, 0:1] * xc[:, 0:1])
    alpha = jnp.maximum(alpha, 1.0 + 1e-7)
    al = alpha.reshape(_BE // D, D)
    sq = jnp.sqrt(jnp.maximum(al * al - 1.0, 1e-12))
    dal = jnp.log(al + sq)
    fac = dal / sq
    d = dal.reshape(_BE, 1)
    w1c0 = wv_ref[0:1, :]
    w1c1 = wv_ref[1:2, :]
    b1 = wv_ref[2:3, :]
    w2 = wv_ref[3:4, :]
    b2 = wv_ref[4:5, 0:1]
    pre = ar + bc + ea_ref[...] * w1c0 + d * w1c1 + b1
    hmid = pre * _sigmoid(pre)
    lg = _rowsum(hmid * w2) + b2
    att_fac = _sigmoid(lg.reshape(_BE // D, D)) * fac
    fa = att_fac.reshape(_BE, 1) * em_ref[...]
    agg_ref[...] = fa * (xc - alpha * xr)


def _epilogue_body(x_ref, o0_ref, o1_ref, ln_ref, out_ref):
    x = x_ref[...]
    first = _first_mask(x.shape)
    out = (o0_ref[...] + o1_ref[...]) * (1.0 / 1000.0)
    lxo = _rowsum(x * out) - 2.0 * x[:, 0:1] * out[:, 0:1]
    p = out + lxo * x
    lpp = _rowsum(p * p) - 2.0 * p[:, 0:1] * p[:, 0:1]
    npn = jnp.sqrt(jnp.maximum(lpp, 1e-12))
    ch, sh = _cosh_sinh(npn)
    x2 = ch * x + (sh / npn) * p
    ht = _logmap0_cols(x2, first)
    mu = _rowsum(ht) * (1.0 / (D - 1))
    dif = jnp.where(first, 0.0, ht - mu)
    var = _rowsum(dif * dif) * (1.0 / (D - 1))
    gamma = ln_ref[0:1, :]
    beta = ln_ref[1:2, :]
    h1 = dif / jnp.sqrt(var + 1e-5) * gamma + beta
    x3 = _expmap0_cols(h1, first)
    s = _logmap0_cols(x3, first)
    sl = s * _sigmoid(s)
    out_ref[...] = _expmap0_cols(sl, first)


_NC = 2
_NS = 16
_NW = _NC * _NS
_EPERW = 10240
_EPAD = _NW * _EPERW
_S = 2
_ESL = _EPAD // _S
_EPWS = _ESL // _NW
_CHG = 80
_NCHG = _EPWS // _CHG
_CHS = 128
_NCHS = _EPWS // _CHS
_NPAD = 10240
_ROWS_PER_TILE = _NPAD // _NS


def _sc_gather(tr, tcb, row3d, col3d):
    mesh = plsc.VectorSubcoreMesh(core_axis_name="c", subcore_axis_name="s")

    @functools.partial(
        pl.kernel, mesh=mesh,
        out_type=[jax.ShapeDtypeStruct((_ESL, D), jnp.int32),
                  jax.ShapeDtypeStruct((_ESL, D), jnp.int32)],
        scratch_types=[
            pltpu.VMEM((_NCHG, _CHG), jnp.int32),
            pltpu.VMEM((_NCHG, _CHG), jnp.int32),
            pltpu.VMEM((_CHG, D), jnp.int32),
            pltpu.VMEM((_CHG, D), jnp.int32),
            pltpu.VMEM((_CHG, D), jnp.int32),
            pltpu.VMEM((_CHG, D), jnp.int32),
            pltpu.SemaphoreType.DMA,
            pltpu.SemaphoreType.DMA,
            pltpu.SemaphoreType.DMA,
            pltpu.SemaphoreType.DMA,
        ],
    )
    def k(tr_hbm, tc_hbm, row_hbm, col_hbm, gr_hbm, gc_hbm,
          idxr, idxc, br0, br1, bc0, bc1, sr0, sr1, sc0, sc1):
        cid = lax.axis_index("c")
        sid = lax.axis_index("s")
        wid = sid * _NC + cid
        bufr = (br0, br1)
        bufc = (bc0, bc1)
        semr = (sr0, sr1)
        semc = (sc0, sc1)
        pltpu.sync_copy(row_hbm.at[wid], idxr)
        pltpu.sync_copy(col_hbm.at[wid], idxc)

        for b in range(2):
            pltpu.async_copy(tr_hbm.at[idxr.at[b]], bufr[b], semr[b])
            pltpu.async_copy(tc_hbm.at[idxc.at[b]], bufc[b], semc[b])

        def body(g, carry):
            for b in range(2):
                j = g * 2 + b
                ebase = pl.multiple_of(wid * _EPWS + j * _CHG, 8)
                pltpu.make_async_copy(tr_hbm.at[idxr.at[j]], bufr[b],
                                      semr[b]).wait()
                pltpu.sync_copy(bufr[b], gr_hbm.at[pl.ds(ebase, _CHG)])
                pltpu.make_async_copy(tc_hbm.at[idxc.at[j]], bufc[b],
                                      semc[b]).wait()
                pltpu.sync_copy(bufc[b], gc_hbm.at[pl.ds(ebase, _CHG)])

                @pl.when(j + 2 < _NCHG)
                def _():
                    pltpu.async_copy(tr_hbm.at[idxr.at[j + 2]], bufr[b],
                                     semr[b])
                    pltpu.async_copy(tc_hbm.at[idxc.at[j + 2]], bufc[b],
                                     semc[b])
            return carry

        lax.fori_loop(0, _NCHG // 2, body, 0)

    return k(tr, tcb, row3d, col3d)


def _sc_scatter(aggs, row3d, zeros_hbm):
    mesh = plsc.VectorSubcoreMesh(core_axis_name="c", subcore_axis_name="s")

    @functools.partial(
        pl.kernel, mesh=mesh,
        out_type=jax.ShapeDtypeStruct((_NC, _NPAD, D), _F32),
        scratch_types=[
            pltpu.VMEM((_S * _NCHS, _CHS), jnp.int32),
            pltpu.VMEM((_CHS, D), _F32),
            pltpu.VMEM((_CHS, D), _F32),
            pltpu.VMEM_SHARED((_NPAD, D), _F32),
            pltpu.SemaphoreType.DMA,
            pltpu.SemaphoreType.DMA,
        ],
    )
    def k(*args):
        agg_refs = args[:_S]
        (row_hbm, zeros_h, parts_hbm, idxr, b0, b1, acc, sm0, sm1) = args[_S:]
        cid = lax.axis_index("c")
        sid = lax.axis_index("s")
        wid = sid * _NC + cid
        buf = (b0, b1)
        sems = (sm0, sm1)
        pltpu.sync_copy(zeros_h, b0)

        def zbody(t, carry):
            rbase = pl.multiple_of(sid * _ROWS_PER_TILE + t * _CHS, 8)
            pltpu.sync_copy(b0, acc.at[pl.ds(rbase, _CHS)])
            return carry

        lax.fori_loop(0, _ROWS_PER_TILE // _CHS, zbody, 0)
        plsc.subcore_barrier()

        pltpu.sync_copy(row_hbm.at[wid], idxr)

        def ebase_of(j):
            return pl.multiple_of(wid * _EPWS + j * _CHS, 8)

        for s in range(_S):
            agg_hbm = agg_refs[s]
            for b in range(2):
                pltpu.async_copy(agg_hbm.at[pl.ds(ebase_of(b), _CHS)],
                                 buf[b], sems[b])

            def body(g, carry, agg_hbm=agg_hbm, s=s):
                for b in range(2):
                    j = g * 2 + b
                    pltpu.make_async_copy(
                        agg_hbm.at[pl.ds(ebase_of(j), _CHS)],
                        buf[b], sems[b]).wait()
                    pltpu.sync_copy(buf[b], acc.at[idxr.at[s * _NCHS + j]],
                                    add=True)

                    @pl.when(j + 2 < _NCHS)
                    def _():
                        pltpu.async_copy(
                            agg_hbm.at[pl.ds(ebase_of(j + 2), _CHS)],
                            buf[b], sems[b])
                return carry

            lax.fori_loop(0, _NCHS // 2, body, 0)
        plsc.subcore_barrier()

        def wbody(t, carry):
            rbase = pl.multiple_of(sid * _ROWS_PER_TILE + t * _CHS, 8)
            pltpu.sync_copy(acc.at[pl.ds(rbase, _CHS)], b0)
            pltpu.sync_copy(b0, parts_hbm.at[cid].at[pl.ds(rbase, _CHS)])
            return carry

        lax.fori_loop(0, _ROWS_PER_TILE // _CHS, wbody, 0)

    return k(*aggs, row3d, zeros_hbm)


_BN = 2000
_BE = 2048


def kernel(h, edge_attr, edges, node_mask, edge_mask, W, bias, att_w1,
           att_b1, att_w2, att_b2, ln_gamma, ln_beta):
    del node_mask
    f32 = _F32
    h = h.astype(f32)
    npad = _EPAD - E
    pad_idx = (jnp.arange(npad, dtype=jnp.int32) * 13) % N
    row = jnp.concatenate([edges[0].astype(jnp.int32), pad_idx])
    col = jnp.concatenate([edges[1].astype(jnp.int32), pad_idx])
    ea_p = jnp.concatenate([edge_attr.astype(f32),
                            jnp.zeros((npad, 1), f32)])
    em_p = jnp.concatenate([edge_mask.astype(f32),
                            jnp.zeros((npad, 1), f32)])
    row_g = row.reshape(_S, _NW, _NCHG, _CHG)
    col_g = col.reshape(_S, _NW, _NCHG, _CHG)
    row_sc = (row.reshape(_S, _NW, _NCHS, _CHS)
              .transpose(1, 0, 2, 3).reshape(_NW, _S * _NCHS, _CHS))
    ea4 = ea_p.reshape(_S, _ESL, 1)
    em4 = em_p.reshape(_S, _ESL, 1)

    wt = W.T.astype(f32)
    w1at = att_w1[:, 0:D].T.astype(f32)
    w1bt = att_w1[:, D:2 * D].T.astype(f32)
    bias2 = bias.astype(f32).reshape(1, D)

    wv = jnp.zeros((8, D), f32)
    wv = wv.at[0].set(att_w1[:, 2 * D])
    wv = wv.at[1].set(att_w1[:, 2 * D + 1])
    wv = wv.at[2].set(att_b1)
    wv = wv.at[3].set(att_w2[0])
    wv = wv.at[4, 0].set(att_b2[0])

    ln = jnp.zeros((2, D), f32)
    ln = ln.at[0, 1:].set(ln_gamma)
    ln = ln.at[1, 1:].set(ln_beta)

    nblk = N // _BN
    x, tr, tcb = pl.pallas_call(
        _prologue_body,
        grid=(nblk,),
        in_specs=[
            pl.BlockSpec((_BN, D), lambda i: (i, 0)),
            pl.BlockSpec((D, D), lambda i: (0, 0)),
            pl.BlockSpec((1, D), lambda i: (0, 0)),
            pl.BlockSpec((D, D), lambda i: (0, 0)),
            pl.BlockSpec((D, D), lambda i: (0, 0)),
        ],
        out_specs=[
            pl.BlockSpec((_BN, D), lambda i: (i, 0)),
            pl.BlockSpec((_BN, D), lambda i: (i, 0)),
            pl.BlockSpec((_BN, D), lambda i: (i, 0)),
        ],
        out_shape=[
            jax.ShapeDtypeStruct((N, D), f32),
            jax.ShapeDtypeStruct((N, D), jnp.int32),
            jax.ShapeDtypeStruct((N, D), jnp.int32),
        ],
    )(h, wt, bias2, w1at, w1bt)

    eblk = _ESL // _BE
    aggs = []
    for s in range(_S):
        gr, gc = _sc_gather(tr, tcb, row_g[s], col_g[s])
        agg_s = pl.pallas_call(
            _edge_body,
            grid=(eblk,),
            in_specs=[
                pl.BlockSpec((_BE, D), lambda i: (i, 0)),
                pl.BlockSpec((_BE, D), lambda i: (i, 0)),
                pl.BlockSpec((_BE, 1), lambda i: (i, 0)),
                pl.BlockSpec((_BE, 1), lambda i: (i, 0)),
                pl.BlockSpec((8, D), lambda i: (0, 0)),
            ],
            out_specs=pl.BlockSpec((_BE, D), lambda i: (i, 0)),
            out_shape=jax.ShapeDtypeStruct((_ESL, D), f32),
        )(gr, gc, ea4[s], em4[s], wv)
        aggs.append(agg_s)

    zeros_h = jnp.zeros((_CHS, D), f32)
    parts = _sc_scatter(aggs, row_sc, zeros_h)
    p0 = parts[0, :N]
    p1 = parts[1, :N]

    out = pl.pallas_call(
        _epilogue_body,
        grid=(nblk,),
        in_specs=[
            pl.BlockSpec((_BN, D), lambda i: (i, 0)),
            pl.BlockSpec((_BN, D), lambda i: (i, 0)),
            pl.BlockSpec((_BN, D), lambda i: (i, 0)),
            pl.BlockSpec((2, D), lambda i: (0, 0)),
        ],
        out_specs=pl.BlockSpec((_BN, D), lambda i: (i, 0)),
        out_shape=jax.ShapeDtypeStruct((N, D), f32),
    )(x, p0, p1, ln)

    return out


# speedup vs baseline: 1.0402x; 1.0113x over previous
import functools

import jax
import jax.numpy as jnp
from jax import lax
from jax.experimental import pallas as pl
from jax.experimental.pallas import tpu as pltpu
from jax.experimental.pallas import tpu_sc as plsc

N = 10000
E = 320000
D = 128

_F32 = jnp.float32


def _acosh(z):
    return jnp.log(z + jnp.sqrt(z * z - 1.0))


def _cosh_sinh(n):
    e = jnp.exp(n)
    ei = 1.0 / e
    return 0.5 * (e + ei), 0.5 * (e - ei)


def _sigmoid(z):
    return 1.0 / (1.0 + jnp.exp(-z))


def _rowsum(z):
    return jnp.sum(z, axis=-1, keepdims=True)


def _first_mask(shape):
    return lax.broadcasted_iota(jnp.int32, shape, len(shape) - 1) == 0


def _logmap0_cols(x, first):
    x0 = jnp.maximum(x[:, 0:1], 1.0 + 1e-7)
    d = _acosh(x0)
    nsq = _rowsum(x * x) - x[:, 0:1] * x[:, 0:1]
    n = jnp.sqrt(jnp.maximum(nsq, 1e-12))
    return jnp.where(first, 0.0, (d / n) * x)


def _expmap0_cols(u, first):
    us = jnp.where(first, 0.0, u)
    nsq = _rowsum(us * us)
    n = jnp.sqrt(jnp.maximum(nsq, 1e-12))
    c, s = _cosh_sinh(n)
    return jnp.where(first, c, (s / n) * us)


def _bf16_bits(x):
    u = lax.bitcast_convert_type(x, jnp.int32)
    return lax.shift_right_logical(
        u + 0x7FFF + (lax.shift_right_logical(u, 16) & 1), 16)


def _pack2(lo, hi):
    return _bf16_bits(lo) | (_bf16_bits(hi) << 16)


def _unpack_lo(w):
    return lax.bitcast_convert_type(w << 16, _F32)


def _unpack_hi(w):
    return lax.bitcast_convert_type(w & jnp.int32(-65536), _F32)


def _prologue_body(h_ref, wt_ref, bias_ref, w1at_ref, w1bt_ref,
                   x_ref, tr_ref, tc_ref):
    h = h_ref[...]
    first = _first_mask(h.shape)
    u = _logmap0_cols(h, first)
    xt = jnp.dot(u, wt_ref[...], preferred_element_type=_F32)
    xt = jnp.where(first, 0.0, xt)
    x = _expmap0_cols(xt, first)
    bmask = jnp.where(_first_mask(bias_ref[...].shape), 0.0, bias_ref[...])
    c = _rowsum(x * bmask) / (1.0 + x[:, 0:1])
    b = bmask + c * (jnp.where(first, 1.0, 0.0) + x)
    lbb = _rowsum(b * b) - 2.0 * b[:, 0:1] * b[:, 0:1]
    nb = jnp.sqrt(jnp.maximum(lbb, 1e-12))
    ch, sh = _cosh_sinh(nb)
    x = ch * x + (sh / nb) * b
    x_ref[...] = x
    x_tan = _logmap0_cols(x, first)
    a = jnp.dot(x_tan, w1at_ref[...], preferred_element_type=_F32)
    bb = jnp.dot(x_tan, w1bt_ref[...], preferred_element_type=_F32)
    tr_ref[...] = _pack2(x, a)
    tc_ref[...] = _pack2(x, bb)


def _edge_body(gr_ref, gc_ref, ea_ref, em_ref, wv_ref, agg_ref):
    wr = gr_ref[...]
    wc = gc_ref[...]
    xr = _unpack_lo(wr)
    ar = _unpack_hi(wr)
    xc = _unpack_lo(wc)
    bc = _unpack_hi(wc)
    alpha = -(_rowsum(xr * xc) - 2.0 * xr[:, 0:1] * xc[:, 0:1])
    alpha = jnp.maximum(alpha, 1.0 + 1e-7)
    al = alpha.reshape(_BE // D, D)
    sq = jnp.sqrt(jnp.maximum(al * al - 1.0, 1e-12))
    dal = jnp.log(al + sq)
    fac = dal / sq
    d = dal.reshape(_BE, 1)
    w1c0 = wv_ref[0:1, :]
    w1c1 = wv_ref[1:2, :]
    b1 = wv_ref[2:3, :]
    w2 = wv_ref[3:4, :]
    b2 = wv_ref[4:5, 0:1]
    pre = ar + bc + ea_ref[...] * w1c0 + d * w1c1 + b1
    hmid = pre * _sigmoid(pre)
    lg = _rowsum(hmid * w2) + b2
    att_fac = _sigmoid(lg.reshape(_BE // D, D)) * fac
    fa = att_fac.reshape(_BE, 1) * em_ref[...]
    agg_ref[...] = fa * (xc - alpha * xr)


def _epilogue_body(x_ref, o0_ref, o1_ref, ln_ref, out_ref):
    x = x_ref[...]
    first = _first_mask(x.shape)
    out = (o0_ref[...] + o1_ref[...]) * (1.0 / 1000.0)
    lxo = _rowsum(x * out) - 2.0 * x[:, 0:1] * out[:, 0:1]
    p = out + lxo * x
    lpp = _rowsum(p * p) - 2.0 * p[:, 0:1] * p[:, 0:1]
    npn = jnp.sqrt(jnp.maximum(lpp, 1e-12))
    ch, sh = _cosh_sinh(npn)
    x2 = ch * x + (sh / npn) * p
    ht = _logmap0_cols(x2, first)
    mu = _rowsum(ht) * (1.0 / (D - 1))
    dif = jnp.where(first, 0.0, ht - mu)
    var = _rowsum(dif * dif) * (1.0 / (D - 1))
    gamma = ln_ref[0:1, :]
    beta = ln_ref[1:2, :]
    h1 = dif / jnp.sqrt(var + 1e-5) * gamma + beta
    x3 = _expmap0_cols(h1, first)
    s = _logmap0_cols(x3, first)
    sl = s * _sigmoid(s)
    out_ref[...] = _expmap0_cols(sl, first)


_NC = 2
_NS = 16
_NW = _NC * _NS
_EPERW = 10240
_EPAD = _NW * _EPERW
_S = 2
_ESL = _EPAD // _S
_EPWS = _ESL // _NW
_CHG = 128
_NCHG = _EPWS // _CHG
_CHS = 128
_NCHS = _EPWS // _CHS
_NPAD = 10240
_ROWS_PER_TILE = _NPAD // _NS


def _sc_gather(tr, tcb, row3d, col3d):
    mesh = plsc.VectorSubcoreMesh(core_axis_name="c", subcore_axis_name="s")

    @functools.partial(
        pl.kernel, mesh=mesh,
        out_type=[jax.ShapeDtypeStruct((_ESL, D), jnp.int32),
                  jax.ShapeDtypeStruct((_ESL, D), jnp.int32)],
        scratch_types=[
            pltpu.VMEM((_NCHG, _CHG), jnp.int32),
            pltpu.VMEM((_NCHG, _CHG), jnp.int32),
            pltpu.VMEM((_CHG, D), jnp.int32),
            pltpu.VMEM((_CHG, D), jnp.int32),
            pltpu.VMEM((_CHG, D), jnp.int32),
            pltpu.VMEM((_CHG, D), jnp.int32),
            pltpu.SemaphoreType.DMA,
            pltpu.SemaphoreType.DMA,
            pltpu.SemaphoreType.DMA,
            pltpu.SemaphoreType.DMA,
        ],
    )
    def k(tr_hbm, tc_hbm, row_hbm, col_hbm, gr_hbm, gc_hbm,
          idxr, idxc, br0, br1, bc0, bc1, sr0, sr1, sc0, sc1):
        cid = lax.axis_index("c")
        sid = lax.axis_index("s")
        wid = sid * _NC + cid
        bufr = (br0, br1)
        bufc = (bc0, bc1)
        semr = (sr0, sr1)
        semc = (sc0, sc1)
        pltpu.sync_copy(row_hbm.at[wid], idxr)
        pltpu.sync_copy(col_hbm.at[wid], idxc)

        for b in range(2):
            pltpu.async_copy(tr_hbm.at[idxr.at[b]], bufr[b], semr[b])
            pltpu.async_copy(tc_hbm.at[idxc.at[b]], bufc[b], semc[b])

        def body(g, carry):
            for b in range(2):
                j = g * 2 + b
                ebase = pl.multiple_of(wid * _EPWS + j * _CHG, 8)
                pltpu.make_async_copy(tr_hbm.at[idxr.at[j]], bufr[b],
                                      semr[b]).wait()
                pltpu.sync_copy(bufr[b], gr_hbm.at[pl.ds(ebase, _CHG)])
                pltpu.make_async_copy(tc_hbm.at[idxc.at[j]], bufc[b],
                                      semc[b]).wait()
                pltpu.sync_copy(bufc[b], gc_hbm.at[pl.ds(ebase, _CHG)])

                @pl.when(j + 2 < _NCHG)
                def _():
                    pltpu.async_copy(tr_hbm.at[idxr.at[j + 2]], bufr[b],
                                     semr[b])
                    pltpu.async_copy(tc_hbm.at[idxc.at[j + 2]], bufc[b],
                                     semc[b])
            return carry

        lax.fori_loop(0, _NCHG // 2, body, 0)

    return k(tr, tcb, row3d, col3d)


def _sc_scatter(aggs, row3d, zeros_hbm):
    mesh = plsc.VectorSubcoreMesh(core_axis_name="c", subcore_axis_name="s")

    @functools.partial(
        pl.kernel, mesh=mesh,
        out_type=jax.ShapeDtypeStruct((_NC, _NPAD, D), _F32),
        scratch_types=[
            pltpu.VMEM((_S * _NCHS, _CHS), jnp.int32),
            pltpu.VMEM((_CHS, D), _F32),
            pltpu.VMEM((_CHS, D), _F32),
            pltpu.VMEM_SHARED((_NPAD, D), _F32),
            pltpu.SemaphoreType.DMA,
            pltpu.SemaphoreType.DMA,
        ],
    )
    def k(*args):
        agg_refs = args[:_S]
        (row_hbm, zeros_h, parts_hbm, idxr, b0, b1, acc, sm0, sm1) = args[_S:]
        cid = lax.axis_index("c")
        sid = lax.axis_index("s")
        wid = sid * _NC + cid
        buf = (b0, b1)
        sems = (sm0, sm1)
        pltpu.sync_copy(zeros_h, b0)

        def zbody(t, carry):
            rbase = pl.multiple_of(sid * _ROWS_PER_TILE + t * _CHS, 8)
            pltpu.sync_copy(b0, acc.at[pl.ds(rbase, _CHS)])
            return carry

        lax.fori_loop(0, _ROWS_PER_TILE // _CHS, zbody, 0)
        plsc.subcore_barrier()

        pltpu.sync_copy(row_hbm.at[wid], idxr)

        def ebase_of(j):
            return pl.multiple_of(wid * _EPWS + j * _CHS, 8)

        for s in range(_S):
            agg_hbm = agg_refs[s]
            for b in range(2):
                pltpu.async_copy(agg_hbm.at[pl.ds(ebase_of(b), _CHS)],
                                 buf[b], sems[b])

            def body(g, carry, agg_hbm=agg_hbm, s=s):
                for b in range(2):
                    j = g * 2 + b
                    pltpu.make_async_copy(
                        agg_hbm.at[pl.ds(ebase_of(j), _CHS)],
                        buf[b], sems[b]).wait()
                    pltpu.sync_copy(buf[b], acc.at[idxr.at[s * _NCHS + j]],
                                    add=True)

                    @pl.when(j + 2 < _NCHS)
                    def _():
                        pltpu.async_copy(
                            agg_hbm.at[pl.ds(ebase_of(j + 2), _CHS)],
                            buf[b], sems[b])
                return carry

            lax.fori_loop(0, _NCHS // 2, body, 0)
        plsc.subcore_barrier()

        def wbody(t, carry):
            rbase = pl.multiple_of(sid * _ROWS_PER_TILE + t * _CHS, 8)
            pltpu.sync_copy(acc.at[pl.ds(rbase, _CHS)], b0)
            pltpu.sync_copy(b0, parts_hbm.at[cid].at[pl.ds(rbase, _CHS)])
            return carry

        lax.fori_loop(0, _ROWS_PER_TILE // _CHS, wbody, 0)

    return k(*aggs, row3d, zeros_hbm)


_BN = 2000
_BE = 2048


def kernel(h, edge_attr, edges, node_mask, edge_mask, W, bias, att_w1,
           att_b1, att_w2, att_b2, ln_gamma, ln_beta):
    del node_mask
    f32 = _F32
    h = h.astype(f32)
    npad = _EPAD - E
    pad_idx = (jnp.arange(npad, dtype=jnp.int32) * 13) % N
    row = jnp.concatenate([edges[0].astype(jnp.int32), pad_idx])
    col = jnp.concatenate([edges[1].astype(jnp.int32), pad_idx])
    ea_p = jnp.concatenate([edge_attr.astype(f32),
                            jnp.zeros((npad, 1), f32)])
    em_p = jnp.concatenate([edge_mask.astype(f32),
                            jnp.zeros((npad, 1), f32)])
    row_g = row.reshape(_S, _NW, _NCHG, _CHG)
    col_g = col.reshape(_S, _NW, _NCHG, _CHG)
    row_sc = (row.reshape(_S, _NW, _NCHS, _CHS)
              .transpose(1, 0, 2, 3).reshape(_NW, _S * _NCHS, _CHS))
    ea4 = ea_p.reshape(_S, _ESL, 1)
    em4 = em_p.reshape(_S, _ESL, 1)

    wt = W.T.astype(f32)
    w1at = att_w1[:, 0:D].T.astype(f32)
    w1bt = att_w1[:, D:2 * D].T.astype(f32)
    bias2 = bias.astype(f32).reshape(1, D)

    wv = jnp.zeros((8, D), f32)
    wv = wv.at[0].set(att_w1[:, 2 * D])
    wv = wv.at[1].set(att_w1[:, 2 * D + 1])
    wv = wv.at[2].set(att_b1)
    wv = wv.at[3].set(att_w2[0])
    wv = wv.at[4, 0].set(att_b2[0])

    ln = jnp.zeros((2, D), f32)
    ln = ln.at[0, 1:].set(ln_gamma)
    ln = ln.at[1, 1:].set(ln_beta)

    nblk = N // _BN
    x, tr, tcb = pl.pallas_call(
        _prologue_body,
        grid=(nblk,),
        in_specs=[
            pl.BlockSpec((_BN, D), lambda i: (i, 0)),
            pl.BlockSpec((D, D), lambda i: (0, 0)),
            pl.BlockSpec((1, D), lambda i: (0, 0)),
            pl.BlockSpec((D, D), lambda i: (0, 0)),
            pl.BlockSpec((D, D), lambda i: (0, 0)),
        ],
        out_specs=[
            pl.BlockSpec((_BN, D), lambda i: (i, 0)),
            pl.BlockSpec((_BN, D), lambda i: (i, 0)),
            pl.BlockSpec((_BN, D), lambda i: (i, 0)),
        ],
        out_shape=[
            jax.ShapeDtypeStruct((N, D), f32),
            jax.ShapeDtypeStruct((N, D), jnp.int32),
            jax.ShapeDtypeStruct((N, D), jnp.int32),
        ],
    )(h, wt, bias2, w1at, w1bt)

    eblk = _ESL // _BE
    aggs = []
    for s in range(_S):
        gr, gc = _sc_gather(tr, tcb, row_g[s], col_g[s])
        agg_s = pl.pallas_call(
            _edge_body,
            grid=(eblk,),
            in_specs=[
                pl.BlockSpec((_BE, D), lambda i: (i, 0)),
                pl.BlockSpec((_BE, D), lambda i: (i, 0)),
                pl.BlockSpec((_BE, 1), lambda i: (i, 0)),
                pl.BlockSpec((_BE, 1), lambda i: (i, 0)),
                pl.BlockSpec((8, D), lambda i: (0, 0)),
            ],
            out_specs=pl.BlockSpec((_BE, D), lambda i: (i, 0)),
            out_shape=jax.ShapeDtypeStruct((_ESL, D), f32),
        )(gr, gc, ea4[s], em4[s], wv)
        aggs.append(agg_s)

    zeros_h = jnp.zeros((_CHS, D), f32)
    parts = _sc_scatter(aggs, row_sc, zeros_h)
    p0 = parts[0, :N]
    p1 = parts[1, :N]

    out = pl.pallas_call(
        _epilogue_body,
        grid=(nblk,),
        in_specs=[
            pl.BlockSpec((_BN, D), lambda i: (i, 0)),
            pl.BlockSpec((_BN, D), lambda i: (i, 0)),
            pl.BlockSpec((_BN, D), lambda i: (i, 0)),
            pl.BlockSpec((2, D), lambda i: (0, 0)),
        ],
        out_specs=pl.BlockSpec((_BN, D), lambda i: (i, 0)),
        out_shape=jax.ShapeDtypeStruct((N, D), f32),
    )(x, p0, p1, ln)

    return out
